# baseline (speedup 1.0000x reference)
import jax
import jax.numpy as jnp
from jax.experimental import pallas as pl


def _mul_body(s_ref, p_ref, o_ref):
    b = pl.program_id(0)
    p = p_ref[b, :]
    o_ref[...] = s_ref[...] * p[None, :, None]


def kernel(sampled_probs, parent_probs, sample_k, sample_min_prob):
    B, L, V = sampled_probs.shape
    K = 64
    global_probs = pl.pallas_call(
        _mul_body,
        grid=(B,),
        in_specs=[
            pl.BlockSpec((1, L, V), lambda b: (b, 0, 0)),
            pl.BlockSpec((B, L), lambda b: (0, 0)),
        ],
        out_specs=pl.BlockSpec((1, L, V), lambda b: (b, 0, 0)),
        out_shape=jax.ShapeDtypeStruct((B, L, V), jnp.float32),
    )(sampled_probs, parent_probs)
    flat = global_probs.reshape(B, L * V)
    topk_probs, topk_indices = jax.lax.top_k(flat, K)
    parent_indices = (topk_indices // V).astype(jnp.int64)
    token_ids = (topk_indices % V).astype(jnp.int64)
    return (token_ids, topk_probs, parent_indices)


# speedup vs baseline: 17.6225x; 17.6225x over previous
import jax
import jax.numpy as jnp
from jax import lax
from jax.experimental import pallas as pl
from jax.experimental.pallas import tpu as pltpu
from jax.experimental.pallas import tpu_sc as plsc

B = 8
NL = 32
V = 100000
K = 64
W1 = 5120
NSLAB = 19
WT = 2720
GROUP = 1280
GPL = 79
RTAIL = 78
TAILW = V - RTAIL * GROUP
GPB = NL * GPL
GPW = 8 * GPL
NSUM = GPB // 16
CAP = 2048
BIGI = 2**30
F32 = jnp.float32
I32 = jnp.int32


def _scbody(sampled, parent, tok_out, prob_out, par_out,
            slab, stail, gwin, gtail, parent_v, gmax_v, gm_v, sum_v, ids_v,
            tmp16f, ids16_v, cnt_v, cand_val_v, cand_idx_v, csum_v,
            mval_v, midx_v, msum_v, l2_v, tok_v, prob_v, par_v,
            gmax_sh, ids_sh, t_sh, val_sh, idx_sh, sum_sh):
    c = lax.axis_index("c")
    s = lax.axis_index("s")
    bl = s // 4
    part = s % 4
    batch = c * 4 + bl
    ix16 = lax.iota(I32, 16)

    def _parent_at(lf):
        pvreg = parent_v[pl.ds((lf // 16) * 16, 16)]
        return jnp.max(jnp.where(ix16 == lf % 16, pvreg, -1.0))

    pltpu.sync_copy(parent.at[pl.ds(batch * NL, NL)], parent_v)

    def group_scan(bufref, lfl, coff, nv, p):
        def vb(j, acc):
            x = bufref[lfl, pl.ds(coff + j * 16, 16)]
            return jnp.maximum(acc, x * p)
        return jnp.max(lax.fori_loop(0, nv, vb, jnp.full((16,), -1.0, F32)))

    def emit(gt, m, gvec):
        gvec = jnp.where(ix16 == gt % 16, m, gvec)

        @pl.when(gt % 16 == 15)
        def _():
            gmax_v[pl.ds((gt // 16) * 16, 16)] = gvec

        return gvec

    def slab_body(ch, gvec):
        pltpu.sync_copy(
            sampled.at[batch, pl.ds(part * 8, 8), pl.ds(ch * W1, W1)], slab)

        def rin_body(r_in, gvec):
            def lf_body(lfl, gvec):
                p = _parent_at(part * 8 + lfl)
                m = group_scan(slab, lfl, r_in * GROUP, GROUP // 16, p)
                return emit((ch * 4 + r_in) * 8 + lfl, m, gvec)

            return lax.fori_loop(0, 8, lf_body, gvec)

        return lax.fori_loop(0, 4, rin_body, gvec)

    gvec = lax.fori_loop(0, NSLAB, slab_body, jnp.full((16,), -2.0, F32))

    pltpu.sync_copy(
        sampled.at[batch, pl.ds(part * 8, 8), pl.ds(NSLAB * W1, WT)], stail)
    for r_in, nv in ((0, GROUP // 16), (1, GROUP // 16), (2, TAILW // 16)):
        def lf_body(lfl, gvec, r_in=r_in, nv=nv):
            p = _parent_at(part * 8 + lfl)
            m = group_scan(stail, lfl, r_in * GROUP, nv, p)
            return emit((NSLAB * 4 + r_in) * 8 + lfl, m, gvec)

        gvec = lax.fori_loop(0, 8, lf_body, gvec)

    gmax_v[pl.ds((GPW // 16) * 16, 16)] = jnp.where(ix16 < GPW % 16, gvec,
                                                    -2.0)
    pltpu.sync_copy(gmax_v.at[pl.ds(0, GPW)],
                    gmax_sh.at[pl.ds(bl * GPB + part * GPW, GPW)])
    plsc.subcore_barrier()

    @pl.when(part == 0)
    def _phase2():
        pltpu.sync_copy(gmax_sh.at[pl.ds(bl * GPB, GPB)], gm_v)
        nsv = (NSUM + 15) // 16
        for sv in range(nsv):
            sum_v[pl.ds(sv * 16, 16)] = jnp.full((16,), -2.0, F32)

        def build(i, carry):
            m = jnp.max(gm_v[pl.ds(i * 16, 16)])
            base = (i // 16) * 16
            cur = sum_v[pl.ds(base, 16)]
            sum_v[pl.ds(base, 16)] = jnp.where(ix16 == (i % 16), m, cur)
            return carry

        lax.fori_loop(0, NSUM, build, 0)

        def extract(k, carry):
            idvec, _t = carry
            macc = sum_v[pl.ds(0, 16)]
            for sv in range(1, nsv):
                macc = jnp.maximum(macc, sum_v[pl.ds(sv * 16, 16)])
            m = jnp.max(macc)
            j0acc = jnp.full((16,), BIGI, I32)
            for sv in range(nsv):
                svv = sum_v[pl.ds(sv * 16, 16)]
                j0acc = jnp.minimum(
                    j0acc, jnp.where(svv == m, sv * 16 + ix16, BIGI))
            j0 = jnp.min(j0acc)
            gv = gm_v[pl.ds(j0 * 16, 16)]
            msk = gv == m
            lane = jnp.min(jnp.where(msk, ix16, BIGI))
            gid = j0 * 16 + lane
            idvec = jnp.where(ix16 == (k % 16), gid, idvec)

            @pl.when(k % 16 == 15)
            def _():
                ids_v[pl.ds((k // 16) * 16, 16)] = idvec

            gv2 = jnp.where(ix16 == lane, -2.0, gv)
            gm_v[pl.ds(j0 * 16, 16)] = gv2
            newm = jnp.max(gv2)
            base = (j0 // 16) * 16
            cur = sum_v[pl.ds(base, 16)]
            sum_v[pl.ds(base, 16)] = jnp.where(ix16 == (j0 % 16), newm, cur)
            return idvec, m

        _, t = lax.fori_loop(0, K, extract,
                             (jnp.full((16,), 0, I32), F32(0.0)))
        pltpu.sync_copy(ids_v, ids_sh.at[pl.ds(bl * K, K)])
        tmp16f[...] = jnp.full((16,), t, F32)
        pltpu.sync_copy(tmp16f, t_sh.at[pl.ds(bl * 16, 16)])

    plsc.subcore_barrier()

    pltpu.sync_copy(ids_sh.at[pl.ds(bl * K + part * 16, 16)], ids16_v)
    pltpu.sync_copy(t_sh.at[pl.ds(bl * 16, 16)], tmp16f)
    t = tmp16f[...][0]

    def initc(i, carry):
        cand_val_v[pl.ds(i * 16, 16)] = jnp.full((16,), -1.0, F32)
        cand_idx_v[pl.ds(i * 16, 16)] = jnp.full((16,), BIGI, I32)
        return carry

    lax.fori_loop(0, CAP // 16, initc, 0)
    cnt_v[...] = jnp.full((16,), 0, I32)

    def comp_scan(bufref, lfl, nv, p, base):
        def vb(j, cv):
            x = bufref[lfl, pl.ds(j * 16, 16)]
            v = x * p
            msk = v >= t
            ones = msk.astype(I32)
            pos = jnp.minimum(cv + plsc.cumsum(ones) - 1, CAP - 1)
            plsc.store_scatter(cand_val_v, [pos], v, mask=msk)
            idxv = base + j * 16 + ix16
            plsc.store_scatter(cand_idx_v, [pos], idxv, mask=msk)
            return cv + plsc.all_reduce_population_count(msk)

        cnt_v[...] = lax.fori_loop(0, nv, vb, cnt_v[...])

    def g_body(i, carry):
        gid = jnp.max(jnp.where(ix16 == i, ids16_v[...], -1))
        ps = gid // GPW
        q = gid % GPW
        r = q // 8
        lfl = q % 8
        lf = ps * 8 + lfl
        p = _parent_at(lf)
        base = lf * V + r * GROUP

        @pl.when(r != RTAIL)
        def _():
            pltpu.sync_copy(
                sampled.at[batch, pl.ds(ps * 8, 8), pl.ds(r * GROUP, GROUP)],
                gwin)
            comp_scan(gwin, lfl, GROUP // 16, p, base)

        @pl.when(r == RTAIL)
        def _():
            pltpu.sync_copy(
                sampled.at[batch, pl.ds(ps * 8, 8),
                           pl.ds(RTAIL * GROUP, TAILW)], gtail)
            comp_scan(gtail, lfl, TAILW // 16, p, base)

        return carry

    lax.fori_loop(0, 16, g_body, 0)

    def sb(i, carry):
        m = jnp.max(cand_val_v[pl.ds(i * 16, 16)])
        base = (i // 16) * 16
        cur = csum_v[pl.ds(base, 16)]
        csum_v[pl.ds(base, 16)] = jnp.where(ix16 == (i % 16), m, cur)
        return carry

    lax.fori_loop(0, CAP // 16, sb, 0)
    pltpu.sync_copy(cand_val_v, val_sh.at[pl.ds(s * CAP, CAP)])
    pltpu.sync_copy(cand_idx_v, idx_sh.at[pl.ds(s * CAP, CAP)])
    pltpu.sync_copy(csum_v, sum_sh.at[pl.ds(s * (CAP // 16), CAP // 16)])
    plsc.subcore_barrier()

    @pl.when(part == 0)
    def _phase4():
        for q in range(4):
            sq = bl * 4 + q
            pltpu.sync_copy(val_sh.at[pl.ds(sq * CAP, CAP)], mval_v.at[pl.ds(q * CAP, CAP)])
            pltpu.sync_copy(idx_sh.at[pl.ds(sq * CAP, CAP)], midx_v.at[pl.ds(q * CAP, CAP)])
            pltpu.sync_copy(sum_sh.at[pl.ds(sq * (CAP // 16), CAP // 16)],
                            msum_v.at[pl.ds(q * (CAP // 16), CAP // 16)])
        nsv = 4 * CAP // 256

        def l2b(i, carry):
            m = jnp.max(msum_v[pl.ds(i * 16, 16)])
            base = (i // 16) * 16
            cur = l2_v[pl.ds(base, 16)]
            l2_v[pl.ds(base, 16)] = jnp.where(ix16 == (i % 16), m, cur)
            return carry

        lax.fori_loop(0, nsv, l2b, 0)

        def extract(k, carry):
            tvec, pvec, prvec = carry
            macc = jnp.maximum(l2_v[pl.ds(0, 16)], l2_v[pl.ds(16, 16)])
            m = jnp.max(macc)
            j1acc = jnp.full((16,), BIGI, I32)
            for sv in range(2):
                lv = l2_v[pl.ds(sv * 16, 16)]
                j1acc = jnp.minimum(
                    j1acc, jnp.where(lv == m, sv * 16 + ix16, BIGI))
            j1 = jnp.min(j1acc)
            svv = msum_v[pl.ds(j1 * 16, 16)]
            j0 = jnp.min(jnp.where(svv == m, j1 * 16 + ix16, BIGI))
            vv = mval_v[pl.ds(j0 * 16, 16)]
            iv = midx_v[pl.ds(j0 * 16, 16)]
            msk = vv == m
            chosen = jnp.min(jnp.where(msk, iv, BIGI))
            lane = jnp.min(jnp.where(msk & (iv == chosen), ix16, BIGI))
            tok = chosen % V
            par = chosen // V
            tvec = jnp.where(ix16 == (k % 16), tok, tvec)
            prvec = jnp.where(ix16 == (k % 16), par, prvec)
            pvec = jnp.where(ix16 == (k % 16), m, pvec)

            @pl.when(k % 16 == 15)
            def _():
                kb = (k // 16) * 16
                tok_v[pl.ds(kb, 16)] = tvec
                prob_v[pl.ds(kb, 16)] = pvec
                par_v[pl.ds(kb, 16)] = prvec

            vv2 = jnp.where(ix16 == lane, -2.0, vv)
            mval_v[pl.ds(j0 * 16, 16)] = vv2
            newm = jnp.max(vv2)
            ls = j0 % 16
            svv2 = jnp.where(ix16 == ls, newm, svv)
            msum_v[pl.ds(j1 * 16, 16)] = svv2
            newm2 = jnp.max(svv2)
            base = (j1 // 16) * 16
            cur = l2_v[pl.ds(base, 16)]
            l2_v[pl.ds(base, 16)] = jnp.where(ix16 == (j1 % 16), newm2, cur)
            return tvec, pvec, prvec

        z16i = jnp.full((16,), 0, I32)
        lax.fori_loop(0, K, extract, (z16i, jnp.full((16,), 0.0, F32), z16i))
        pltpu.sync_copy(tok_v, tok_out.at[pl.ds(batch * K, K)])
        pltpu.sync_copy(prob_v, prob_out.at[pl.ds(batch * K, K)])
        pltpu.sync_copy(par_v, par_out.at[pl.ds(batch * K, K)])

    plsc.subcore_barrier()


@jax.jit
def _sc_topk(sampled_probs, parent_probs):
    mesh = plsc.VectorSubcoreMesh(core_axis_name="c", subcore_axis_name="s",
                                  num_cores=2, num_subcores=16)
    f = pl.kernel(
        _scbody,
        out_type=(
            jax.ShapeDtypeStruct((B * K,), I32),
            jax.ShapeDtypeStruct((B * K,), F32),
            jax.ShapeDtypeStruct((B * K,), I32),
        ),
        mesh=mesh,
        compiler_params=pltpu.CompilerParams(needs_layout_passes=False),
        scratch_types=[
            pltpu.VMEM((8, W1), F32),
            pltpu.VMEM((8, WT), F32),
            pltpu.VMEM((8, GROUP), F32),
            pltpu.VMEM((8, TAILW), F32),
            pltpu.VMEM((NL,), F32),
            pltpu.VMEM((GPW + 8,), F32),
            pltpu.VMEM((GPB,), F32),
            pltpu.VMEM((160,), F32),
            pltpu.VMEM((K,), I32),
            pltpu.VMEM((16,), F32),
            pltpu.VMEM((16,), I32),
            pltpu.VMEM((16,), I32),
            pltpu.VMEM((CAP,), F32),
            pltpu.VMEM((CAP,), I32),
            pltpu.VMEM((CAP // 16,), F32),
            pltpu.VMEM((4 * CAP,), F32),
            pltpu.VMEM((4 * CAP,), I32),
            pltpu.VMEM((4 * CAP // 16,), F32),
            pltpu.VMEM((32,), F32),
            pltpu.VMEM((K,), I32),
            pltpu.VMEM((K,), F32),
            pltpu.VMEM((K,), I32),
            pltpu.VMEM_SHARED((4 * GPB,), F32),
            pltpu.VMEM_SHARED((4 * K,), I32),
            pltpu.VMEM_SHARED((64,), F32),
            pltpu.VMEM_SHARED((16 * CAP,), F32),
            pltpu.VMEM_SHARED((16 * CAP,), I32),
            pltpu.VMEM_SHARED((16 * (CAP // 16),), F32),
        ],
    )
    return f(sampled_probs, parent_probs)


def kernel(sampled_probs, parent_probs, sample_k, sample_min_prob):
    tok, probs, par = _sc_topk(sampled_probs, parent_probs.reshape(-1))
    tok = tok.reshape(B, K)
    probs = probs.reshape(B, K)
    par = par.reshape(B, K)
    token_ids = tok.astype(jnp.int64)
    parent_indices = par.astype(jnp.int64)
    return (token_ids, probs, parent_indices)


# speedup vs baseline: 42.2819x; 2.3993x over previous
import jax
import jax.numpy as jnp
from jax import lax
from jax.experimental import pallas as pl
from jax.experimental.pallas import tpu as pltpu
from jax.experimental.pallas import tpu_sc as plsc

B = 8
NL = 32
V = 100000
K = 64
W1 = 3840
NSLAB = 26
WT = 160
GROUP = 1280
GPL = 79
RTAIL = 78
TAILW = V - RTAIL * GROUP
GPB = NL * GPL
GPW = 8 * GPL
NSUM = GPB // 16
CAP = 2048
BIGI = 2**30
F32 = jnp.float32
I32 = jnp.int32


def _scbody(sampled, parent, tok_out, prob_out, par_out,
            slab, slabB, stail, gwin, gtail, parent_v, gmax_v, gm_v, sum_v,
            ids_v, tmp16f, ids16_v, cnt_v, cand_val_v, cand_idx_v, csum_v,
            mval_v, midx_v, msum_v, l2_v, tok_v, prob_v, par_v,
            gmax_sh, ids_sh, t_sh, val_sh, idx_sh, sum_sh, semA, semB):
    c = lax.axis_index("c")
    s = lax.axis_index("s")
    bl = s // 4
    part = s % 4
    batch = c * 4 + bl
    ix16 = lax.iota(I32, 16)

    def _parent_at(lf):
        pvreg = parent_v[pl.ds((lf // 16) * 16, 16)]
        return jnp.max(jnp.where(ix16 == lf % 16, pvreg, -1.0))

    pltpu.sync_copy(parent.at[pl.ds(batch * NL, NL)], parent_v)

    def group_scan(bufref, lfl, coff, nv, p):
        def vb(j, acc):
            x = bufref[lfl, pl.ds(coff + j * 16, 16)]
            return jnp.maximum(acc, x * p)
        return jnp.max(lax.fori_loop(0, nv, vb, jnp.full((16,), -1.0, F32),
                                     unroll=10))

    def emit(gt, m, gvec):
        gvec = jnp.where(ix16 == gt % 16, m, gvec)

        @pl.when(gt % 16 == 15)
        def _():
            gmax_v[pl.ds((gt // 16) * 16, 16)] = gvec

        return gvec

    def _src(ch):
        return sampled.at[batch, pl.ds(part * 8, 8), pl.ds(ch * W1, W1)]

    def process(buf, ch, gvec):
        def rin_body(r_in, gvec):
            def lf_body(lfl, gvec):
                p = _parent_at(part * 8 + lfl)
                m = group_scan(buf, lfl, r_in * GROUP, GROUP // 16, p)
                return emit((ch * 3 + r_in) * 8 + lfl, m, gvec)

            return lax.fori_loop(0, 8, lf_body, gvec)

        return lax.fori_loop(0, 3, rin_body, gvec)

    pltpu.async_copy(_src(0), slab, semA)

    def blk_body(b, gvec):
        chA = 2 * b
        chB = chA + 1
        pltpu.async_copy(_src(chB), slabB, semB)
        pltpu.make_async_copy(_src(chA), slab, semA).wait()
        gvec = process(slab, chA, gvec)

        @pl.when(b < NSLAB // 2 - 1)
        def _():
            pltpu.async_copy(_src(chA + 2), slab, semA)

        pltpu.make_async_copy(_src(chB), slabB, semB).wait()
        return process(slabB, chB, gvec)

    gvec = lax.fori_loop(0, NSLAB // 2, blk_body,
                         jnp.full((16,), -2.0, F32))

    pltpu.sync_copy(
        sampled.at[batch, pl.ds(part * 8, 8), pl.ds(NSLAB * W1, WT)], stail)

    def tail_body(lfl, gvec):
        p = _parent_at(part * 8 + lfl)
        m = group_scan(stail, lfl, 0, TAILW // 16, p)
        return emit(NSLAB * 3 * 8 + lfl, m, gvec)

    gvec = lax.fori_loop(0, 8, tail_body, gvec)

    gmax_v[pl.ds((GPW // 16) * 16, 16)] = jnp.where(ix16 < GPW % 16, gvec,
                                                    -2.0)
    pltpu.sync_copy(gmax_v.at[pl.ds(0, GPW)],
                    gmax_sh.at[pl.ds(bl * GPB + part * GPW, GPW)])
    plsc.subcore_barrier()

    @pl.when(part == 0)
    def _phase2():
        pltpu.sync_copy(gmax_sh.at[pl.ds(bl * GPB, GPB)], gm_v)
        nsv = (NSUM + 15) // 16
        for sv in range(nsv):
            sum_v[pl.ds(sv * 16, 16)] = jnp.full((16,), -2.0, F32)

        def build(i, carry):
            m = jnp.max(gm_v[pl.ds(i * 16, 16)])
            base = (i // 16) * 16
            cur = sum_v[pl.ds(base, 16)]
            sum_v[pl.ds(base, 16)] = jnp.where(ix16 == (i % 16), m, cur)
            return carry

        lax.fori_loop(0, NSUM, build, 0, unroll=4)

        def extract(k, carry):
            idvec, _t = carry
            macc = sum_v[pl.ds(0, 16)]
            for sv in range(1, nsv):
                macc = jnp.maximum(macc, sum_v[pl.ds(sv * 16, 16)])
            m = jnp.max(macc)
            j0acc = jnp.full((16,), BIGI, I32)
            for sv in range(nsv):
                svv = sum_v[pl.ds(sv * 16, 16)]
                j0acc = jnp.minimum(
                    j0acc, jnp.where(svv == m, sv * 16 + ix16, BIGI))
            j0 = jnp.min(j0acc)
            gv = gm_v[pl.ds(j0 * 16, 16)]
            msk = gv == m
            lane = jnp.min(jnp.where(msk, ix16, BIGI))
            gid = j0 * 16 + lane
            idvec = jnp.where(ix16 == (k % 16), gid, idvec)

            @pl.when(k % 16 == 15)
            def _():
                ids_v[pl.ds((k // 16) * 16, 16)] = idvec

            gv2 = jnp.where(ix16 == lane, -2.0, gv)
            gm_v[pl.ds(j0 * 16, 16)] = gv2
            newm = jnp.max(gv2)
            base = (j0 // 16) * 16
            cur = sum_v[pl.ds(base, 16)]
            sum_v[pl.ds(base, 16)] = jnp.where(ix16 == (j0 % 16), newm, cur)
            return idvec, m

        _, t = lax.fori_loop(0, K, extract,
                             (jnp.full((16,), 0, I32), F32(0.0)))
        pltpu.sync_copy(ids_v, ids_sh.at[pl.ds(bl * K, K)])
        tmp16f[...] = jnp.full((16,), t, F32)
        pltpu.sync_copy(tmp16f, t_sh.at[pl.ds(bl * 16, 16)])

    plsc.subcore_barrier()

    pltpu.sync_copy(ids_sh.at[pl.ds(bl * K + part * 16, 16)], ids16_v)
    pltpu.sync_copy(t_sh.at[pl.ds(bl * 16, 16)], tmp16f)
    t = tmp16f[...][0]

    def initc(i, carry):
        cand_val_v[pl.ds(i * 16, 16)] = jnp.full((16,), -1.0, F32)
        cand_idx_v[pl.ds(i * 16, 16)] = jnp.full((16,), BIGI, I32)
        return carry

    lax.fori_loop(0, CAP // 16, initc, 0, unroll=8)
    cnt_v[...] = jnp.full((16,), 0, I32)

    def comp_scan(bufref, lfl, nv, p, base):
        def vb(j, cv):
            x = bufref[lfl, pl.ds(j * 16, 16)]
            v = x * p
            msk = v >= t
            ones = msk.astype(I32)
            pos = jnp.minimum(cv + plsc.cumsum(ones) - 1, CAP - 1)
            plsc.store_scatter(cand_val_v, [pos], v, mask=msk)
            idxv = base + j * 16 + ix16
            plsc.store_scatter(cand_idx_v, [pos], idxv, mask=msk)
            return cv + plsc.all_reduce_population_count(msk)

        cnt_v[...] = lax.fori_loop(0, nv, vb, cnt_v[...], unroll=4)

    def g_body(i, carry):
        gid = jnp.max(jnp.where(ix16 == i, ids16_v[...], -1))
        ps = gid // GPW
        q = gid % GPW
        r = q // 8
        lfl = q % 8
        lf = ps * 8 + lfl
        p = _parent_at(lf)
        base = lf * V + r * GROUP

        @pl.when(r != RTAIL)
        def _():
            pltpu.sync_copy(
                sampled.at[batch, pl.ds(ps * 8, 8), pl.ds(r * GROUP, GROUP)],
                gwin)
            comp_scan(gwin, lfl, GROUP // 16, p, base)

        @pl.when(r == RTAIL)
        def _():
            pltpu.sync_copy(
                sampled.at[batch, pl.ds(ps * 8, 8),
                           pl.ds(RTAIL * GROUP, TAILW)], gtail)
            comp_scan(gtail, lfl, TAILW // 16, p, base)

        return carry

    lax.fori_loop(0, 16, g_body, 0)

    def sb(i, carry):
        m = jnp.max(cand_val_v[pl.ds(i * 16, 16)])
        base = (i // 16) * 16
        cur = csum_v[pl.ds(base, 16)]
        csum_v[pl.ds(base, 16)] = jnp.where(ix16 == (i % 16), m, cur)
        return carry

    lax.fori_loop(0, CAP // 16, sb, 0, unroll=4)
    pltpu.sync_copy(cand_val_v, val_sh.at[pl.ds(s * CAP, CAP)])
    pltpu.sync_copy(cand_idx_v, idx_sh.at[pl.ds(s * CAP, CAP)])
    pltpu.sync_copy(csum_v, sum_sh.at[pl.ds(s * (CAP // 16), CAP // 16)])
    plsc.subcore_barrier()

    @pl.when(part == 0)
    def _phase4():
        for q in range(4):
            sq = bl * 4 + q
            pltpu.sync_copy(val_sh.at[pl.ds(sq * CAP, CAP)], mval_v.at[pl.ds(q * CAP, CAP)])
            pltpu.sync_copy(idx_sh.at[pl.ds(sq * CAP, CAP)], midx_v.at[pl.ds(q * CAP, CAP)])
            pltpu.sync_copy(sum_sh.at[pl.ds(sq * (CAP // 16), CAP // 16)],
                            msum_v.at[pl.ds(q * (CAP // 16), CAP // 16)])
        nsv = 4 * CAP // 256

        def l2b(i, carry):
            m = jnp.max(msum_v[pl.ds(i * 16, 16)])
            base = (i // 16) * 16
            cur = l2_v[pl.ds(base, 16)]
            l2_v[pl.ds(base, 16)] = jnp.where(ix16 == (i % 16), m, cur)
            return carry

        lax.fori_loop(0, nsv, l2b, 0, unroll=4)

        def extract(k, carry):
            tvec, pvec, prvec = carry
            macc = jnp.maximum(l2_v[pl.ds(0, 16)], l2_v[pl.ds(16, 16)])
            m = jnp.max(macc)
            j1acc = jnp.full((16,), BIGI, I32)
            for sv in range(2):
                lv = l2_v[pl.ds(sv * 16, 16)]
                j1acc = jnp.minimum(
                    j1acc, jnp.where(lv == m, sv * 16 + ix16, BIGI))
            j1 = jnp.min(j1acc)
            svv = msum_v[pl.ds(j1 * 16, 16)]
            j0 = jnp.min(jnp.where(svv == m, j1 * 16 + ix16, BIGI))
            vv = mval_v[pl.ds(j0 * 16, 16)]
            iv = midx_v[pl.ds(j0 * 16, 16)]
            msk = vv == m
            chosen = jnp.min(jnp.where(msk, iv, BIGI))
            lane = jnp.min(jnp.where(msk & (iv == chosen), ix16, BIGI))
            tok = chosen % V
            par = chosen // V
            tvec = jnp.where(ix16 == (k % 16), tok, tvec)
            prvec = jnp.where(ix16 == (k % 16), par, prvec)
            pvec = jnp.where(ix16 == (k % 16), m, pvec)

            @pl.when(k % 16 == 15)
            def _():
                kb = (k // 16) * 16
                tok_v[pl.ds(kb, 16)] = tvec
                prob_v[pl.ds(kb, 16)] = pvec
                par_v[pl.ds(kb, 16)] = prvec

            vv2 = jnp.where(ix16 == lane, -2.0, vv)
            mval_v[pl.ds(j0 * 16, 16)] = vv2
            newm = jnp.max(vv2)
            ls = j0 % 16
            svv2 = jnp.where(ix16 == ls, newm, svv)
            msum_v[pl.ds(j1 * 16, 16)] = svv2
            newm2 = jnp.max(svv2)
            base = (j1 // 16) * 16
            cur = l2_v[pl.ds(base, 16)]
            l2_v[pl.ds(base, 16)] = jnp.where(ix16 == (j1 % 16), newm2, cur)
            return tvec, pvec, prvec

        z16i = jnp.full((16,), 0, I32)
        lax.fori_loop(0, K, extract, (z16i, jnp.full((16,), 0.0, F32), z16i))
        pltpu.sync_copy(tok_v, tok_out.at[pl.ds(batch * K, K)])
        pltpu.sync_copy(prob_v, prob_out.at[pl.ds(batch * K, K)])
        pltpu.sync_copy(par_v, par_out.at[pl.ds(batch * K, K)])

    plsc.subcore_barrier()


@jax.jit
def _sc_topk(sampled_probs, parent_probs):
    mesh = plsc.VectorSubcoreMesh(core_axis_name="c", subcore_axis_name="s",
                                  num_cores=2, num_subcores=16)
    f = pl.kernel(
        _scbody,
        out_type=(
            jax.ShapeDtypeStruct((B * K,), I32),
            jax.ShapeDtypeStruct((B * K,), F32),
            jax.ShapeDtypeStruct((B * K,), I32),
        ),
        mesh=mesh,
        compiler_params=pltpu.CompilerParams(needs_layout_passes=False),
        scratch_types=[
            pltpu.VMEM((8, W1), F32),
            pltpu.VMEM((8, W1), F32),
            pltpu.VMEM((8, WT), F32),
            pltpu.VMEM((8, GROUP), F32),
            pltpu.VMEM((8, TAILW), F32),
            pltpu.VMEM((NL,), F32),
            pltpu.VMEM((GPW + 8,), F32),
            pltpu.VMEM((GPB,), F32),
            pltpu.VMEM((160,), F32),
            pltpu.VMEM((K,), I32),
            pltpu.VMEM((16,), F32),
            pltpu.VMEM((16,), I32),
            pltpu.VMEM((16,), I32),
            pltpu.VMEM((CAP,), F32),
            pltpu.VMEM((CAP,), I32),
            pltpu.VMEM((CAP // 16,), F32),
            pltpu.VMEM((4 * CAP,), F32),
            pltpu.VMEM((4 * CAP,), I32),
            pltpu.VMEM((4 * CAP // 16,), F32),
            pltpu.VMEM((32,), F32),
            pltpu.VMEM((K,), I32),
            pltpu.VMEM((K,), F32),
            pltpu.VMEM((K,), I32),
            pltpu.VMEM_SHARED((4 * GPB,), F32),
            pltpu.VMEM_SHARED((4 * K,), I32),
            pltpu.VMEM_SHARED((64,), F32),
            pltpu.VMEM_SHARED((16 * CAP,), F32),
            pltpu.VMEM_SHARED((16 * CAP,), I32),
            pltpu.VMEM_SHARED((16 * (CAP // 16),), F32),
            pltpu.SemaphoreType.DMA,
            pltpu.SemaphoreType.DMA,
        ],
    )
    return f(sampled_probs, parent_probs)


def kernel(sampled_probs, parent_probs, sample_k, sample_min_prob):
    tok, probs, par = _sc_topk(sampled_probs, parent_probs.reshape(-1))
    tok = tok.reshape(B, K)
    probs = probs.reshape(B, K)
    par = par.reshape(B, K)
    token_ids = tok.astype(jnp.int64)
    parent_indices = par.astype(jnp.int64)
    return (token_ids, probs, parent_indices)


# speedup vs baseline: 44.9233x; 1.0625x over previous
import jax
import jax.numpy as jnp
from jax import lax
from jax.experimental import pallas as pl
from jax.experimental.pallas import tpu as pltpu
from jax.experimental.pallas import tpu_sc as plsc

B = 8
NL = 32
V = 100000
K = 64
W1 = 3840
NSLAB = 26
WT = 160
GROUP = 1280
GPL = 79
RTAIL = 78
TAILW = V - RTAIL * GROUP
GPB = NL * GPL
GPW = 8 * GPL
NSUM = GPB // 16
CAP = 2048
BIGI = 2**30
F32 = jnp.float32
I32 = jnp.int32


def _scbody(sampled, parent, tok_out, prob_out, par_out,
            slab, slabB, stail, gwin, gtail, parent_v, gmax_v, gm_v, sum_v,
            ids_v, tmp16f, ids16_v, cnt_v, cand_val_v, cand_idx_v, csum_v,
            mval_v, midx_v, msum_v, l2_v, tok_v, prob_v, par_v,
            gmax_sh, ids_sh, t_sh, val_sh, idx_sh, sum_sh, semA, semB):
    c = lax.axis_index("c")
    s = lax.axis_index("s")
    bl = s // 4
    part = s % 4
    batch = c * 4 + bl
    ix16 = lax.iota(I32, 16)

    def _parent_at(lf):
        pvreg = parent_v[pl.ds((lf // 16) * 16, 16)]
        return jnp.max(jnp.where(ix16 == lf % 16, pvreg, -1.0))

    pltpu.sync_copy(parent.at[pl.ds(batch * NL, NL)], parent_v)

    def group_scan(bufref, lfl, coff, nv, p):
        neg = jnp.full((16,), -1.0, F32)

        def vb4(j, accs):
            a0, a1, a2, a3 = accs
            base = coff + j * 64
            a0 = jnp.maximum(a0, bufref[lfl, pl.ds(base, 16)])
            a1 = jnp.maximum(a1, bufref[lfl, pl.ds(base + 16, 16)])
            a2 = jnp.maximum(a2, bufref[lfl, pl.ds(base + 32, 16)])
            a3 = jnp.maximum(a3, bufref[lfl, pl.ds(base + 48, 16)])
            return a0, a1, a2, a3

        def vb1(j, acc):
            return jnp.maximum(acc, bufref[lfl, pl.ds(coff + j * 16, 16)])

        if nv % 4 == 0:
            accs = lax.fori_loop(0, nv // 4, vb4, (neg, neg, neg, neg),
                                 unroll=5)
            acc = jnp.maximum(jnp.maximum(accs[0], accs[1]),
                              jnp.maximum(accs[2], accs[3]))
        else:
            acc = lax.fori_loop(0, nv, vb1, neg, unroll=10)
        return p * jnp.max(acc)

    def emit(gt, m, gvec):
        gvec = jnp.where(ix16 == gt % 16, m, gvec)

        @pl.when(gt % 16 == 15)
        def _():
            gmax_v[pl.ds((gt // 16) * 16, 16)] = gvec

        return gvec

    def _src(ch):
        return sampled.at[batch, pl.ds(part * 8, 8), pl.ds(ch * W1, W1)]

    def process(buf, ch, gvec):
        def rin_body(r_in, gvec):
            def lf_body(lfl, gvec):
                p = _parent_at(part * 8 + lfl)
                m = group_scan(buf, lfl, r_in * GROUP, GROUP // 16, p)
                return emit((ch * 3 + r_in) * 8 + lfl, m, gvec)

            return lax.fori_loop(0, 8, lf_body, gvec)

        return lax.fori_loop(0, 3, rin_body, gvec)

    pltpu.async_copy(_src(0), slab, semA)

    def blk_body(b, gvec):
        chA = 2 * b
        chB = chA + 1
        pltpu.async_copy(_src(chB), slabB, semB)
        pltpu.make_async_copy(_src(chA), slab, semA).wait()
        gvec = process(slab, chA, gvec)

        @pl.when(b < NSLAB // 2 - 1)
        def _():
            pltpu.async_copy(_src(chA + 2), slab, semA)

        pltpu.make_async_copy(_src(chB), slabB, semB).wait()
        return process(slabB, chB, gvec)

    gvec = lax.fori_loop(0, NSLAB // 2, blk_body,
                         jnp.full((16,), -2.0, F32))

    pltpu.sync_copy(
        sampled.at[batch, pl.ds(part * 8, 8), pl.ds(NSLAB * W1, WT)], stail)

    def tail_body(lfl, gvec):
        p = _parent_at(part * 8 + lfl)
        m = group_scan(stail, lfl, 0, TAILW // 16, p)
        return emit(NSLAB * 3 * 8 + lfl, m, gvec)

    gvec = lax.fori_loop(0, 8, tail_body, gvec)

    gmax_v[pl.ds((GPW // 16) * 16, 16)] = jnp.where(ix16 < GPW % 16, gvec,
                                                    -2.0)
    pltpu.sync_copy(gmax_v.at[pl.ds(0, GPW)],
                    gmax_sh.at[pl.ds(bl * GPB + part * GPW, GPW)])
    plsc.subcore_barrier()

    @pl.when(part == 0)
    def _phase2():
        pltpu.sync_copy(gmax_sh.at[pl.ds(bl * GPB, GPB)], gm_v)
        nsv = (NSUM + 15) // 16
        for sv in range(nsv):
            sum_v[pl.ds(sv * 16, 16)] = jnp.full((16,), -2.0, F32)

        def build(i, carry):
            m = jnp.max(gm_v[pl.ds(i * 16, 16)])
            base = (i // 16) * 16
            cur = sum_v[pl.ds(base, 16)]
            sum_v[pl.ds(base, 16)] = jnp.where(ix16 == (i % 16), m, cur)
            return carry

        lax.fori_loop(0, NSUM, build, 0, unroll=4)

        def extract(k, carry):
            idvec, _t = carry
            macc = sum_v[pl.ds(0, 16)]
            for sv in range(1, nsv):
                macc = jnp.maximum(macc, sum_v[pl.ds(sv * 16, 16)])
            m = jnp.max(macc)
            j0acc = jnp.full((16,), BIGI, I32)
            for sv in range(nsv):
                svv = sum_v[pl.ds(sv * 16, 16)]
                j0acc = jnp.minimum(
                    j0acc, jnp.where(svv == m, sv * 16 + ix16, BIGI))
            j0 = jnp.min(j0acc)
            gv = gm_v[pl.ds(j0 * 16, 16)]
            msk = gv == m
            lane = jnp.min(jnp.where(msk, ix16, BIGI))
            gid = j0 * 16 + lane
            idvec = jnp.where(ix16 == (k % 16), gid, idvec)

            @pl.when(k % 16 == 15)
            def _():
                ids_v[pl.ds((k // 16) * 16, 16)] = idvec

            gv2 = jnp.where(ix16 == lane, -2.0, gv)
            gm_v[pl.ds(j0 * 16, 16)] = gv2
            newm = jnp.max(gv2)
            base = (j0 // 16) * 16
            cur = sum_v[pl.ds(base, 16)]
            sum_v[pl.ds(base, 16)] = jnp.where(ix16 == (j0 % 16), newm, cur)
            return idvec, m

        _, t = lax.fori_loop(0, K, extract,
                             (jnp.full((16,), 0, I32), F32(0.0)))
        pltpu.sync_copy(ids_v, ids_sh.at[pl.ds(bl * K, K)])
        tmp16f[...] = jnp.full((16,), t, F32)
        pltpu.sync_copy(tmp16f, t_sh.at[pl.ds(bl * 16, 16)])

    plsc.subcore_barrier()

    pltpu.sync_copy(ids_sh.at[pl.ds(bl * K + part * 16, 16)], ids16_v)
    pltpu.sync_copy(t_sh.at[pl.ds(bl * 16, 16)], tmp16f)
    t = tmp16f[...][0]

    def initc(i, carry):
        cand_val_v[pl.ds(i * 16, 16)] = jnp.full((16,), -1.0, F32)
        cand_idx_v[pl.ds(i * 16, 16)] = jnp.full((16,), BIGI, I32)
        return carry

    lax.fori_loop(0, CAP // 16, initc, 0, unroll=8)
    cnt_v[...] = jnp.full((16,), 0, I32)

    def comp_scan(bufref, lfl, nv, p, base):
        def vb(j, cv):
            x = bufref[lfl, pl.ds(j * 16, 16)]
            v = x * p
            msk = v >= t
            ones = msk.astype(I32)
            pos = jnp.minimum(cv + plsc.cumsum(ones) - 1, CAP - 1)
            plsc.store_scatter(cand_val_v, [pos], v, mask=msk)
            idxv = base + j * 16 + ix16
            plsc.store_scatter(cand_idx_v, [pos], idxv, mask=msk)
            return cv + plsc.all_reduce_population_count(msk)

        cnt_v[...] = lax.fori_loop(0, nv, vb, cnt_v[...], unroll=4)

    def g_body(i, carry):
        gid = jnp.max(jnp.where(ix16 == i, ids16_v[...], -1))
        ps = gid // GPW
        q = gid % GPW
        r = q // 8
        lfl = q % 8
        lf = ps * 8 + lfl
        p = _parent_at(lf)
        base = lf * V + r * GROUP

        @pl.when(r != RTAIL)
        def _():
            pltpu.sync_copy(
                sampled.at[batch, pl.ds(ps * 8, 8), pl.ds(r * GROUP, GROUP)],
                gwin)
            comp_scan(gwin, lfl, GROUP // 16, p, base)

        @pl.when(r == RTAIL)
        def _():
            pltpu.sync_copy(
                sampled.at[batch, pl.ds(ps * 8, 8),
                           pl.ds(RTAIL * GROUP, TAILW)], gtail)
            comp_scan(gtail, lfl, TAILW // 16, p, base)

        return carry

    lax.fori_loop(0, 16, g_body, 0)

    def sb(i, carry):
        m = jnp.max(cand_val_v[pl.ds(i * 16, 16)])
        base = (i // 16) * 16
        cur = csum_v[pl.ds(base, 16)]
        csum_v[pl.ds(base, 16)] = jnp.where(ix16 == (i % 16), m, cur)
        return carry

    lax.fori_loop(0, CAP // 16, sb, 0, unroll=4)
    pltpu.sync_copy(cand_val_v, val_sh.at[pl.ds(s * CAP, CAP)])
    pltpu.sync_copy(cand_idx_v, idx_sh.at[pl.ds(s * CAP, CAP)])
    pltpu.sync_copy(csum_v, sum_sh.at[pl.ds(s * (CAP // 16), CAP // 16)])
    plsc.subcore_barrier()

    @pl.when(part == 0)
    def _phase4():
        for q in range(4):
            sq = bl * 4 + q
            pltpu.sync_copy(val_sh.at[pl.ds(sq * CAP, CAP)], mval_v.at[pl.ds(q * CAP, CAP)])
            pltpu.sync_copy(idx_sh.at[pl.ds(sq * CAP, CAP)], midx_v.at[pl.ds(q * CAP, CAP)])
            pltpu.sync_copy(sum_sh.at[pl.ds(sq * (CAP // 16), CAP // 16)],
                            msum_v.at[pl.ds(q * (CAP // 16), CAP // 16)])
        nsv = 4 * CAP // 256

        def l2b(i, carry):
            m = jnp.max(msum_v[pl.ds(i * 16, 16)])
            base = (i // 16) * 16
            cur = l2_v[pl.ds(base, 16)]
            l2_v[pl.ds(base, 16)] = jnp.where(ix16 == (i % 16), m, cur)
            return carry

        lax.fori_loop(0, nsv, l2b, 0, unroll=4)

        def extract(k, carry):
            tvec, pvec, prvec = carry
            macc = jnp.maximum(l2_v[pl.ds(0, 16)], l2_v[pl.ds(16, 16)])
            m = jnp.max(macc)
            j1acc = jnp.full((16,), BIGI, I32)
            for sv in range(2):
                lv = l2_v[pl.ds(sv * 16, 16)]
                j1acc = jnp.minimum(
                    j1acc, jnp.where(lv == m, sv * 16 + ix16, BIGI))
            j1 = jnp.min(j1acc)
            svv = msum_v[pl.ds(j1 * 16, 16)]
            j0 = jnp.min(jnp.where(svv == m, j1 * 16 + ix16, BIGI))
            vv = mval_v[pl.ds(j0 * 16, 16)]
            iv = midx_v[pl.ds(j0 * 16, 16)]
            msk = vv == m
            chosen = jnp.min(jnp.where(msk, iv, BIGI))
            lane = jnp.min(jnp.where(msk & (iv == chosen), ix16, BIGI))
            tok = chosen % V
            par = chosen // V
            tvec = jnp.where(ix16 == (k % 16), tok, tvec)
            prvec = jnp.where(ix16 == (k % 16), par, prvec)
            pvec = jnp.where(ix16 == (k % 16), m, pvec)

            @pl.when(k % 16 == 15)
            def _():
                kb = (k // 16) * 16
                tok_v[pl.ds(kb, 16)] = tvec
                prob_v[pl.ds(kb, 16)] = pvec
                par_v[pl.ds(kb, 16)] = prvec

            vv2 = jnp.where(ix16 == lane, -2.0, vv)
            mval_v[pl.ds(j0 * 16, 16)] = vv2
            newm = jnp.max(vv2)
            ls = j0 % 16
            svv2 = jnp.where(ix16 == ls, newm, svv)
            msum_v[pl.ds(j1 * 16, 16)] = svv2
            newm2 = jnp.max(svv2)
            base = (j1 // 16) * 16
            cur = l2_v[pl.ds(base, 16)]
            l2_v[pl.ds(base, 16)] = jnp.where(ix16 == (j1 % 16), newm2, cur)
            return tvec, pvec, prvec

        z16i = jnp.full((16,), 0, I32)
        lax.fori_loop(0, K, extract, (z16i, jnp.full((16,), 0.0, F32), z16i))
        pltpu.sync_copy(tok_v, tok_out.at[pl.ds(batch * K, K)])
        pltpu.sync_copy(prob_v, prob_out.at[pl.ds(batch * K, K)])
        pltpu.sync_copy(par_v, par_out.at[pl.ds(batch * K, K)])

    plsc.subcore_barrier()


@jax.jit
def _sc_topk(sampled_probs, parent_probs):
    mesh = plsc.VectorSubcoreMesh(core_axis_name="c", subcore_axis_name="s",
                                  num_cores=2, num_subcores=16)
    f = pl.kernel(
        _scbody,
        out_type=(
            jax.ShapeDtypeStruct((B * K,), I32),
            jax.ShapeDtypeStruct((B * K,), F32),
            jax.ShapeDtypeStruct((B * K,), I32),
        ),
        mesh=mesh,
        compiler_params=pltpu.CompilerParams(needs_layout_passes=False),
        scratch_types=[
            pltpu.VMEM((8, W1), F32),
            pltpu.VMEM((8, W1), F32),
            pltpu.VMEM((8, WT), F32),
            pltpu.VMEM((8, GROUP), F32),
            pltpu.VMEM((8, TAILW), F32),
            pltpu.VMEM((NL,), F32),
            pltpu.VMEM((GPW + 8,), F32),
            pltpu.VMEM((GPB,), F32),
            pltpu.VMEM((160,), F32),
            pltpu.VMEM((K,), I32),
            pltpu.VMEM((16,), F32),
            pltpu.VMEM((16,), I32),
            pltpu.VMEM((16,), I32),
            pltpu.VMEM((CAP,), F32),
            pltpu.VMEM((CAP,), I32),
            pltpu.VMEM((CAP // 16,), F32),
            pltpu.VMEM((4 * CAP,), F32),
            pltpu.VMEM((4 * CAP,), I32),
            pltpu.VMEM((4 * CAP // 16,), F32),
            pltpu.VMEM((32,), F32),
            pltpu.VMEM((K,), I32),
            pltpu.VMEM((K,), F32),
            pltpu.VMEM((K,), I32),
            pltpu.VMEM_SHARED((4 * GPB,), F32),
            pltpu.VMEM_SHARED((4 * K,), I32),
            pltpu.VMEM_SHARED((64,), F32),
            pltpu.VMEM_SHARED((16 * CAP,), F32),
            pltpu.VMEM_SHARED((16 * CAP,), I32),
            pltpu.VMEM_SHARED((16 * (CAP // 16),), F32),
            pltpu.SemaphoreType.DMA,
            pltpu.SemaphoreType.DMA,
        ],
    )
    return f(sampled_probs, parent_probs)


def kernel(sampled_probs, parent_probs, sample_k, sample_min_prob):
    tok, probs, par = _sc_topk(sampled_probs, parent_probs.reshape(-1))
    tok = tok.reshape(B, K)
    probs = probs.reshape(B, K)
    par = par.reshape(B, K)
    token_ids = tok.astype(jnp.int64)
    parent_indices = par.astype(jnp.int64)
    return (token_ids, probs, parent_indices)
